# baseline (speedup 1.0000x reference)
import functools

import jax
import jax.numpy as jnp
from jax import lax
from jax.experimental import pallas as pl
from jax.experimental.pallas import tpu as pltpu
from jax.experimental.pallas import tpu_sc as plsc

NC = 2
NS = 16
NW = NC * NS
CHUNK = 128
BLK = 16
FLUSH = 2048
BUFC = 3456
TRASH = 313

_MESH = dict(core_axis_name="c", subcore_axis_name="s",
             num_cores=NC, num_subcores=NS)


def _node_lo(w):
    return 312 * w + jnp.minimum(w, 16)


def _make_bucket(n_nodes, n_edges):
    seg = n_edges + BUFC
    scn = 1280
    nscan = n_edges // scn

    @functools.partial(
        pl.kernel,
        mesh=plsc.VectorSubcoreMesh(**_MESH),
        compiler_params=pltpu.CompilerParams(use_tc_tiling_on_sc=False, needs_layout_passes=False),
        out_type=(
            jax.ShapeDtypeStruct((NW * seg,), jnp.int32),
            jax.ShapeDtypeStruct((NW * seg,), jnp.int32),
            jax.ShapeDtypeStruct((NW * seg,), jnp.int32),
            jax.ShapeDtypeStruct((NW * 8 + 16,), jnp.int32),
        ),
        scratch_types=[
            pltpu.VMEM((scn,), jnp.int32),
            pltpu.VMEM((scn,), jnp.int32),
            pltpu.VMEM((BUFC,), jnp.int32),
            pltpu.VMEM((BUFC,), jnp.int32),
            pltpu.VMEM((BUFC,), jnp.int32),
            pltpu.VMEM((16,), jnp.int32),
        ],
    )
    def bucket_kernel(src_hbm, dst_hbm, bsrc_hbm, bdl_hbm, beid_hbm, cnt_hbm,
                      src_v, dst_v, bs_v, bd_v, be_v, cv):
        c = lax.axis_index("c")
        s = lax.axis_index("s")
        w = c * NS + s
        lo = _node_lo(w)
        hi = _node_lo(w + 1)
        base = w * seg
        lanes = lax.iota(jnp.int32, 16)

        def flush(written, amt):
            off = pl.multiple_of(base + written, 8)
            pltpu.sync_copy(bs_v.at[pl.ds(0, amt)],
                            bsrc_hbm.at[pl.ds(off, amt)])
            pltpu.sync_copy(bd_v.at[pl.ds(0, amt)],
                            bdl_hbm.at[pl.ds(off, amt)])
            pltpu.sync_copy(be_v.at[pl.ds(0, amt)],
                            beid_hbm.at[pl.ds(off, amt)])

        def scan_chunk(ci, carry):
            ptr_v, written = carry
            coff = pl.multiple_of(ci * scn, 8)
            pltpu.sync_copy(src_hbm.at[pl.ds(coff, scn)], src_v)
            pltpu.sync_copy(dst_hbm.at[pl.ds(coff, scn)], dst_v)

            def group(g, ptr_v):
                sl = pl.ds(g * 16, 16)
                dvec = dst_v[sl]
                svec = src_v[sl]
                eid = ci * scn + g * 16 + lanes
                mask = (dvec >= lo) & (dvec < hi)
                pc = plsc.cumsum(mask.astype(jnp.int32))
                idx = ptr_v + pc - 1
                plsc.store_scatter(bs_v, [idx], svec, mask=mask)
                plsc.store_scatter(bd_v, [idx], dvec - lo, mask=mask)
                plsc.store_scatter(be_v, [idx], eid, mask=mask)
                return ptr_v + plsc.all_reduce_population_count(mask)

            ptr_v = lax.fori_loop(0, scn // 16, group, ptr_v, unroll=2)

            def do_flush(args):
                ptr_v, written = args
                flush(written, FLUSH)
                for b in (bs_v, bd_v, be_v):
                    for t in range((BUFC - FLUSH) // 16):
                        tail = b[pl.ds(FLUSH + t * 16, 16)]
                        b[pl.ds(t * 16, 16)] = tail
                return ptr_v - FLUSH, written + FLUSH

            return lax.cond(ptr_v[0] >= FLUSH, do_flush, lambda a: a,
                            (ptr_v, written))

        ptr_v, written = lax.fori_loop(0, nscan, scan_chunk,
                                       (jnp.zeros((16,), jnp.int32),
                                        jnp.int32(0)))
        ptr = ptr_v[0]

        total = written + ptr
        pad = (-total) % CHUNK
        zeros16 = jnp.zeros((16,), jnp.int32)
        trash16 = jnp.full((16,), TRASH, jnp.int32)
        lanes16 = lax.iota(jnp.int32, 16)
        for t in range(CHUNK // 16):
            pidx = ptr + t * 16 + lanes16
            plsc.store_scatter(bs_v, [pidx], zeros16)
            plsc.store_scatter(bd_v, [pidx], trash16)
            plsc.store_scatter(be_v, [pidx], zeros16)
        flush(written, BUFC)
        cv[pl.ds(0, 16)] = jnp.full((16,), total + pad, jnp.int32)
        pltpu.sync_copy(cv.at[pl.ds(0, 8)],
                        cnt_hbm.at[pl.ds(pl.multiple_of(w * 8, 8), 8)])

    return bucket_kernel, seg


def _make_edge_aggr(n_nodes, n_edges, d, seg):
    @functools.partial(
        pl.kernel,
        mesh=plsc.VectorSubcoreMesh(**_MESH),
        compiler_params=pltpu.CompilerParams(use_tc_tiling_on_sc=False, needs_layout_passes=False),
        out_type=jax.ShapeDtypeStruct((n_nodes, d), jnp.float32),
        scratch_types=[
            pltpu.VMEM((2, BLK * CHUNK), jnp.int32),
            pltpu.VMEM((2, BLK * CHUNK), jnp.int32),
            pltpu.VMEM((2, BLK * CHUNK), jnp.int32),
            pltpu.VMEM((2, CHUNK, d), jnp.float32),
            pltpu.VMEM((2, CHUNK, d), jnp.float32),
            pltpu.VMEM((314, d), jnp.float32),
            pltpu.VMEM((NW * 8 + 16,), jnp.int32),
            pltpu.SemaphoreType.DMA,
            pltpu.SemaphoreType.DMA,
            pltpu.SemaphoreType.DMA,
            pltpu.SemaphoreType.DMA,
        ],
    )
    def edge_kernel(h_hbm, e_hbm, bsrc_hbm, bdl_hbm, beid_hbm, cnt_hbm,
                    out_hbm, srcb, dlb, eidb, rows_v, e_v, aggr, cv,
                    bsem0, bsem1, gsem0, gsem1):
        c = lax.axis_index("c")
        s = lax.axis_index("s")
        w = c * NS + s
        lo = _node_lo(w)
        base = w * seg
        bsems = (bsem0, bsem1)
        gsems = (gsem0, gsem1)

        def zbody(i, _):
            for v in range(d // 16):
                aggr[i, pl.ds(v * 16, 16)] = jnp.zeros((16,), jnp.float32)
            return 0
        lax.fori_loop(0, 314, zbody, 0)

        pltpu.sync_copy(cnt_hbm, cv)
        n = cv[pl.ds(w * 8, 16)][0] // CHUNK
        nb = (n + BLK - 1) // BLK

        def blk_descs(B, bb):
            off = pl.ds(pl.multiple_of(base + B * (BLK * CHUNK), 8), BLK * CHUNK)
            return (
                pltpu.make_async_copy(bsrc_hbm.at[off], srcb.at[bb], bsems[bb]),
                pltpu.make_async_copy(bdl_hbm.at[off], dlb.at[bb], bsems[bb]),
                pltpu.make_async_copy(beid_hbm.at[off], eidb.at[bb], bsems[bb]),
            )

        def issue_blk(B, bb):
            for dsc in blk_descs(B, bb):
                dsc.start()

        def wait_blk(B, bb):
            for dsc in blk_descs(B, bb):
                dsc.wait()

        def g_descs(bb, j, g, i):
            isl = pl.ds(pl.multiple_of(j * CHUNK, 8), CHUNK)
            return (
                pltpu.make_async_copy(h_hbm.at[srcb.at[bb, isl]],
                                      rows_v.at[g], gsems[g]),
                pltpu.make_async_copy(e_hbm.at[eidb.at[bb, isl]],
                                      e_v.at[g], gsems[g]),
            )

        def issue_g(bb, j, g, i):
            for dsc in g_descs(bb, j, g, i):
                dsc.start()

        def wait_g(bb, j, g, i):
            for dsc in g_descs(bb, j, g, i):
                dsc.wait()

        @pl.when(nb > 0)
        def _():
            issue_blk(0, 0)

        def block_body(B, bb):
            wait_blk(B, bb)

            @pl.when(B + 1 < nb)
            def _():
                issue_blk(B + 1, 1 - bb)

            @pl.when(B * BLK < n)
            def _():
                issue_g(bb, 0, 0, B * BLK)

            def pair(t, _):
                for parity in range(2):
                    j = 2 * t + parity
                    i = B * BLK + j

                    @pl.when(i < n)
                    def _():
                        @pl.when((j + 1 < BLK) & (i + 1 < n))
                        def _():
                            issue_g(bb, j + 1, 1 - parity, i + 1)
                        wait_g(bb, j, parity, i)

                        def ebody(g16, _):
                            dvec = dlb[bb, pl.ds(j * CHUNK + g16 * 16, 16)]
                            for kk in range(16):
                                dl = dvec[kk]
                                ii = g16 * 16 + kk
                                for v in range(d // 16):
                                    sl = pl.ds(v * 16, 16)
                                    m = jnp.maximum(
                                        rows_v[parity, ii, sl]
                                        + e_v[parity, ii, sl], 0.0)
                                    plsc.addupdate(aggr.at[dl, sl], m)
                            return 0
                        lax.fori_loop(0, CHUNK // 16, ebody, 0, unroll=2)
                return 0
            lax.fori_loop(0, BLK // 2, pair, 0)

        def outer(t, _):
            for parity in range(2):
                B = 2 * t + parity

                @pl.when(B < nb)
                def _():
                    block_body(B, parity)
            return 0
        lax.fori_loop(0, (nb + 1) // 2, outer, 0)

        @pl.when(w < 16)
        def _():
            pltpu.sync_copy(aggr.at[pl.ds(0, 313)],
                            out_hbm.at[pl.ds(lo, 313)])

        @pl.when(w >= 16)
        def _():
            pltpu.sync_copy(aggr.at[pl.ds(0, 312)],
                            out_hbm.at[pl.ds(lo, 312)])

    return edge_kernel


def _dot16(a, b):
    return jnp.dot(a.astype(jnp.bfloat16), b.astype(jnp.bfloat16),
                   preferred_element_type=jnp.float32)


def _edge_bias(edge_attr, we, be):
    e_total, k = edge_attr.shape
    d = we.shape[1]
    be2 = be.reshape(1, d)
    blk = 4000

    def body(a_ref, w_ref, b_ref, o_ref):
        o_ref[...] = _dot16(a_ref[...], w_ref[...]) + b_ref[...]

    return pl.pallas_call(
        body,
        grid=(e_total // blk,),
        in_specs=[
            pl.BlockSpec((blk, k), lambda i: (i, 0)),
            pl.BlockSpec((k, d), lambda i: (0, 0)),
            pl.BlockSpec((1, d), lambda i: (0, 0)),
        ],
        out_specs=pl.BlockSpec((blk, d), lambda i: (i, 0)),
        out_shape=jax.ShapeDtypeStruct((e_total, d), jnp.float32),
    )(edge_attr, we, be2)


def _node_mlp(h, a0, w1, b1, w2, b2):
    n, din = h.shape
    dh = w1.shape[1]
    blk = 1000

    def body(h_ref, a0_ref, w1_ref, b1_ref, w2_ref, b2_ref, o_ref):
        z = h_ref[...] + a0_ref[...]
        t = jnp.maximum(_dot16(z, w1_ref[...]) + b1_ref[...], 0.0)
        o_ref[...] = jnp.maximum(_dot16(t, w2_ref[...]) + b2_ref[...], 0.0)

    return pl.pallas_call(
        body,
        grid=(n // blk,),
        in_specs=[
            pl.BlockSpec((blk, din), lambda i: (i, 0)),
            pl.BlockSpec((blk, din), lambda i: (i, 0)),
            pl.BlockSpec((din, dh), lambda i: (0, 0)),
            pl.BlockSpec((1, dh), lambda i: (0, 0)),
            pl.BlockSpec((dh, dh), lambda i: (0, 0)),
            pl.BlockSpec((1, dh), lambda i: (0, 0)),
        ],
        out_specs=pl.BlockSpec((blk, dh), lambda i: (i, 0)),
        out_shape=jax.ShapeDtypeStruct((n, dh), jnp.float32),
    )(h, a0, w1, b1.reshape(1, dh), w2, b2.reshape(1, dh))


def _head(h, w1, b1, w2, b2, w3, b3):
    n, d1 = h.shape
    d2, d3, dout = w1.shape[1], w2.shape[1], w3.shape[1]
    blk = 1000

    def body(h_ref, w1_ref, b1_ref, w2_ref, b2_ref, w3_ref, b3_ref, o_ref):
        t = jnp.maximum(_dot16(h_ref[...], w1_ref[...]) + b1_ref[...], 0.0)
        t = jnp.maximum(_dot16(t, w2_ref[...]) + b2_ref[...], 0.0)
        o_ref[...] = _dot16(t, w3_ref[...]) + b3_ref[...]

    return pl.pallas_call(
        body,
        grid=(n // blk,),
        in_specs=[
            pl.BlockSpec((blk, d1), lambda i: (i, 0)),
            pl.BlockSpec((d1, d2), lambda i: (0, 0)),
            pl.BlockSpec((1, d2), lambda i: (0, 0)),
            pl.BlockSpec((d2, d3), lambda i: (0, 0)),
            pl.BlockSpec((1, d3), lambda i: (0, 0)),
            pl.BlockSpec((d3, dout), lambda i: (0, 0)),
            pl.BlockSpec((1, dout), lambda i: (0, 0)),
        ],
        out_specs=pl.BlockSpec((blk, dout), lambda i: (i, 0)),
        out_shape=jax.ShapeDtypeStruct((n, dout), jnp.float32),
    )(h, w1, b1.reshape(1, d2), w2, b2.reshape(1, d3), w3, b3.reshape(1, dout))


def kernel(x, edge_index, edge_attr, params):
    n, nfeat = x.shape
    e_total = edge_attr.shape[0]
    src = edge_index[0].astype(jnp.int32)
    dst = edge_index[1].astype(jnp.int32)

    d0 = 16
    h = jnp.pad(x, ((0, 0), (0, d0 - nfeat)))

    bucket, seg = _make_bucket(n, e_total)
    bsrc, bdl, beid, cnts = bucket(src, dst)

    edge16 = _make_edge_aggr(n, e_total, d0, seg)
    edge128 = _make_edge_aggr(n, e_total, 128, seg)

    for i in range(6):
        p = params["conv%d" % i]
        if i == 0:
            we = jnp.pad(p["we"], ((0, 0), (0, d0 - nfeat)))
            be = jnp.pad(p["be"], (0, d0 - nfeat))
            w1 = jnp.pad(p["w1"], ((0, d0 - nfeat), (0, 0)))
            ek = edge16
        else:
            we, be, w1 = p["we"], p["be"], p["w1"]
            ek = edge128
        e = _edge_bias(edge_attr, we, be)
        ag = ek(h, e, bsrc, bdl, beid, cnts)
        h = _node_mlp(h, ag, w1, p["b1"], p["w2"], p["b2"])

    return _head(h, params["lin1_w"], params["lin1_b"],
                 params["lin2_w"], params["lin2_b"],
                 params["lin3_w"], params["lin3_b"])


# speedup vs baseline: 1.1820x; 1.1820x over previous
import functools

import jax
import jax.numpy as jnp
from jax import lax
from jax.experimental import pallas as pl
from jax.experimental.pallas import tpu as pltpu
from jax.experimental.pallas import tpu_sc as plsc

NC = 2
NS = 16
NW = NC * NS
CHUNK = 128
BLK = 16
FLUSH = 2048
BUFC = 3456
TRASH = 313

_MESH = dict(core_axis_name="c", subcore_axis_name="s",
             num_cores=NC, num_subcores=NS)


def _node_lo(w):
    return 312 * w + jnp.minimum(w, 16)


def _make_bucket(n_nodes, n_edges):
    seg = n_edges + BUFC
    scn = 1280
    nscan = n_edges // scn

    @functools.partial(
        pl.kernel,
        mesh=plsc.VectorSubcoreMesh(**_MESH),
        compiler_params=pltpu.CompilerParams(use_tc_tiling_on_sc=False, needs_layout_passes=False),
        out_type=(
            jax.ShapeDtypeStruct((NW * seg,), jnp.int32),
            jax.ShapeDtypeStruct((NW * seg,), jnp.int32),
            jax.ShapeDtypeStruct((NW * seg,), jnp.int32),
            jax.ShapeDtypeStruct((NW * 8 + 16,), jnp.int32),
        ),
        scratch_types=[
            pltpu.VMEM((scn,), jnp.int32),
            pltpu.VMEM((scn,), jnp.int32),
            pltpu.VMEM((BUFC,), jnp.int32),
            pltpu.VMEM((BUFC,), jnp.int32),
            pltpu.VMEM((BUFC,), jnp.int32),
            pltpu.VMEM((16,), jnp.int32),
        ],
    )
    def bucket_kernel(src_hbm, dst_hbm, bsrc_hbm, bdl_hbm, beid_hbm, cnt_hbm,
                      src_v, dst_v, bs_v, bd_v, be_v, cv):
        c = lax.axis_index("c")
        s = lax.axis_index("s")
        w = c * NS + s
        lo = _node_lo(w)
        hi = _node_lo(w + 1)
        base = w * seg
        lanes = lax.iota(jnp.int32, 16)

        def flush(written, amt):
            off = pl.multiple_of(base + written, 8)
            pltpu.sync_copy(bs_v.at[pl.ds(0, amt)],
                            bsrc_hbm.at[pl.ds(off, amt)])
            pltpu.sync_copy(bd_v.at[pl.ds(0, amt)],
                            bdl_hbm.at[pl.ds(off, amt)])
            pltpu.sync_copy(be_v.at[pl.ds(0, amt)],
                            beid_hbm.at[pl.ds(off, amt)])

        def scan_chunk(ci, carry):
            ptr_v, written = carry
            coff = pl.multiple_of(ci * scn, 8)
            pltpu.sync_copy(src_hbm.at[pl.ds(coff, scn)], src_v)
            pltpu.sync_copy(dst_hbm.at[pl.ds(coff, scn)], dst_v)

            def group(g, ptr_v):
                sl = pl.ds(g * 16, 16)
                dvec = dst_v[sl]
                svec = src_v[sl]
                eid = ci * scn + g * 16 + lanes
                mask = (dvec >= lo) & (dvec < hi)
                pc = plsc.cumsum(mask.astype(jnp.int32))
                idx = ptr_v + pc - 1
                plsc.store_scatter(bs_v, [idx], svec, mask=mask)
                plsc.store_scatter(bd_v, [idx], dvec - lo, mask=mask)
                plsc.store_scatter(be_v, [idx], eid, mask=mask)
                return ptr_v + plsc.all_reduce_population_count(mask)

            ptr_v = lax.fori_loop(0, scn // 16, group, ptr_v, unroll=2)

            def do_flush(args):
                ptr_v, written = args
                flush(written, FLUSH)
                for b in (bs_v, bd_v, be_v):
                    for t in range((BUFC - FLUSH) // 16):
                        tail = b[pl.ds(FLUSH + t * 16, 16)]
                        b[pl.ds(t * 16, 16)] = tail
                return ptr_v - FLUSH, written + FLUSH

            return lax.cond(ptr_v[0] >= FLUSH, do_flush, lambda a: a,
                            (ptr_v, written))

        ptr_v, written = lax.fori_loop(0, nscan, scan_chunk,
                                       (jnp.zeros((16,), jnp.int32),
                                        jnp.int32(0)))
        ptr = ptr_v[0]

        total = written + ptr
        pad = (-total) % CHUNK
        zeros16 = jnp.zeros((16,), jnp.int32)
        trash16 = jnp.full((16,), TRASH, jnp.int32)
        lanes16 = lax.iota(jnp.int32, 16)
        for t in range(CHUNK // 16):
            pidx = ptr + t * 16 + lanes16
            plsc.store_scatter(bs_v, [pidx], zeros16)
            plsc.store_scatter(bd_v, [pidx], trash16)
            plsc.store_scatter(be_v, [pidx], zeros16)
        flush(written, BUFC)
        cv[pl.ds(0, 16)] = jnp.full((16,), total + pad, jnp.int32)
        pltpu.sync_copy(cv.at[pl.ds(0, 8)],
                        cnt_hbm.at[pl.ds(pl.multiple_of(w * 8, 8), 8)])

    return bucket_kernel, seg


def _make_edge_aggr(n_nodes, n_edges, d, seg):
    @functools.partial(
        pl.kernel,
        mesh=plsc.VectorSubcoreMesh(**_MESH),
        compiler_params=pltpu.CompilerParams(use_tc_tiling_on_sc=False, needs_layout_passes=False),
        out_type=jax.ShapeDtypeStruct((n_nodes, d), jnp.float32),
        scratch_types=[
            pltpu.VMEM((2, BLK * CHUNK), jnp.int32),
            pltpu.VMEM((2, BLK * CHUNK), jnp.int32),
            pltpu.VMEM((2, BLK * CHUNK), jnp.int32),
            pltpu.VMEM((2, CHUNK, d), jnp.float32),
            pltpu.VMEM((2, CHUNK, d), jnp.float32),
            pltpu.VMEM((314, d), jnp.float32),
            pltpu.VMEM((NW * 8 + 16,), jnp.int32),
            pltpu.SemaphoreType.DMA,
            pltpu.SemaphoreType.DMA,
            pltpu.SemaphoreType.DMA,
            pltpu.SemaphoreType.DMA,
        ],
    )
    def edge_kernel(h_hbm, e_hbm, bsrc_hbm, bdl_hbm, beid_hbm, cnt_hbm,
                    out_hbm, srcb, dlb, eidb, rows_v, e_v, aggr, cv,
                    bsem0, bsem1, gsem0, gsem1):
        c = lax.axis_index("c")
        s = lax.axis_index("s")
        w = c * NS + s
        lo = _node_lo(w)
        base = w * seg
        bsems = (bsem0, bsem1)
        gsems = (gsem0, gsem1)

        def zbody(i, _):
            for v in range(d // 16):
                aggr[i, pl.ds(v * 16, 16)] = jnp.zeros((16,), jnp.float32)
            return 0
        lax.fori_loop(0, 314, zbody, 0)

        pltpu.sync_copy(cnt_hbm, cv)
        n = cv[pl.ds(w * 8, 16)][0] // CHUNK
        nb = (n + BLK - 1) // BLK

        def blk_descs(B, bb):
            off = pl.ds(pl.multiple_of(base + B * (BLK * CHUNK), 8), BLK * CHUNK)
            return (
                pltpu.make_async_copy(bsrc_hbm.at[off], srcb.at[bb], bsems[bb]),
                pltpu.make_async_copy(bdl_hbm.at[off], dlb.at[bb], bsems[bb]),
                pltpu.make_async_copy(beid_hbm.at[off], eidb.at[bb], bsems[bb]),
            )

        def issue_blk(B, bb):
            for dsc in blk_descs(B, bb):
                dsc.start()

        def wait_blk(B, bb):
            for dsc in blk_descs(B, bb):
                dsc.wait()

        def g_descs(bb, j, g, i):
            isl = pl.ds(pl.multiple_of(j * CHUNK, 8), CHUNK)
            return (
                pltpu.make_async_copy(h_hbm.at[srcb.at[bb, isl]],
                                      rows_v.at[g], gsems[g]),
                pltpu.make_async_copy(e_hbm.at[eidb.at[bb, isl]],
                                      e_v.at[g], gsems[g]),
            )

        def issue_g(bb, j, g, i):
            for dsc in g_descs(bb, j, g, i):
                dsc.start()

        def wait_g(bb, j, g, i):
            for dsc in g_descs(bb, j, g, i):
                dsc.wait()

        @pl.when(nb > 0)
        def _():
            issue_blk(0, 0)

        def block_body(B, bb):
            wait_blk(B, bb)

            @pl.when(B + 1 < nb)
            def _():
                issue_blk(B + 1, 1 - bb)

            @pl.when(B * BLK < n)
            def _():
                issue_g(bb, 0, 0, B * BLK)

            def pair(t, _):
                for parity in range(2):
                    j = 2 * t + parity
                    i = B * BLK + j

                    @pl.when(i < n)
                    def _():
                        @pl.when((j + 1 < BLK) & (i + 1 < n))
                        def _():
                            issue_g(bb, j + 1, 1 - parity, i + 1)
                        wait_g(bb, j, parity, i)

                        def ebody(g16, _):
                            dvec = dlb[bb, pl.ds(j * CHUNK + g16 * 16, 16)]
                            for kk in range(16):
                                dl = dvec[kk]
                                ii = g16 * 16 + kk
                                for v in range(d // 16):
                                    sl = pl.ds(v * 16, 16)
                                    m = jnp.maximum(
                                        rows_v[parity, ii, sl]
                                        + e_v[parity, ii, sl], 0.0)
                                    plsc.addupdate(aggr.at[dl, sl], m)
                            return 0
                        lax.fori_loop(0, CHUNK // 16, ebody, 0)
                return 0
            lax.fori_loop(0, BLK // 2, pair, 0)

        def outer(t, _):
            for parity in range(2):
                B = 2 * t + parity

                @pl.when(B < nb)
                def _():
                    block_body(B, parity)
            return 0
        lax.fori_loop(0, (nb + 1) // 2, outer, 0)

        @pl.when(w < 16)
        def _():
            pltpu.sync_copy(aggr.at[pl.ds(0, 313)],
                            out_hbm.at[pl.ds(lo, 313)])

        @pl.when(w >= 16)
        def _():
            pltpu.sync_copy(aggr.at[pl.ds(0, 312)],
                            out_hbm.at[pl.ds(lo, 312)])

    return edge_kernel


def _dot16(a, b):
    return jnp.dot(a.astype(jnp.bfloat16), b.astype(jnp.bfloat16),
                   preferred_element_type=jnp.float32)


def _edge_bias(edge_attr, we, be):
    e_total, k = edge_attr.shape
    d = we.shape[1]
    be2 = be.reshape(1, d)
    blk = 4000

    def body(a_ref, w_ref, b_ref, o_ref):
        o_ref[...] = _dot16(a_ref[...], w_ref[...]) + b_ref[...]

    return pl.pallas_call(
        body,
        grid=(e_total // blk,),
        in_specs=[
            pl.BlockSpec((blk, k), lambda i: (i, 0)),
            pl.BlockSpec((k, d), lambda i: (0, 0)),
            pl.BlockSpec((1, d), lambda i: (0, 0)),
        ],
        out_specs=pl.BlockSpec((blk, d), lambda i: (i, 0)),
        out_shape=jax.ShapeDtypeStruct((e_total, d), jnp.float32),
    )(edge_attr, we, be2)


def _node_mlp(h, a0, w1, b1, w2, b2):
    n, din = h.shape
    dh = w1.shape[1]
    blk = 1000

    def body(h_ref, a0_ref, w1_ref, b1_ref, w2_ref, b2_ref, o_ref):
        z = h_ref[...] + a0_ref[...]
        t = jnp.maximum(_dot16(z, w1_ref[...]) + b1_ref[...], 0.0)
        o_ref[...] = jnp.maximum(_dot16(t, w2_ref[...]) + b2_ref[...], 0.0)

    return pl.pallas_call(
        body,
        grid=(n // blk,),
        in_specs=[
            pl.BlockSpec((blk, din), lambda i: (i, 0)),
            pl.BlockSpec((blk, din), lambda i: (i, 0)),
            pl.BlockSpec((din, dh), lambda i: (0, 0)),
            pl.BlockSpec((1, dh), lambda i: (0, 0)),
            pl.BlockSpec((dh, dh), lambda i: (0, 0)),
            pl.BlockSpec((1, dh), lambda i: (0, 0)),
        ],
        out_specs=pl.BlockSpec((blk, dh), lambda i: (i, 0)),
        out_shape=jax.ShapeDtypeStruct((n, dh), jnp.float32),
    )(h, a0, w1, b1.reshape(1, dh), w2, b2.reshape(1, dh))


def _head(h, w1, b1, w2, b2, w3, b3):
    n, d1 = h.shape
    d2, d3, dout = w1.shape[1], w2.shape[1], w3.shape[1]
    blk = 1000

    def body(h_ref, w1_ref, b1_ref, w2_ref, b2_ref, w3_ref, b3_ref, o_ref):
        t = jnp.maximum(_dot16(h_ref[...], w1_ref[...]) + b1_ref[...], 0.0)
        t = jnp.maximum(_dot16(t, w2_ref[...]) + b2_ref[...], 0.0)
        o_ref[...] = _dot16(t, w3_ref[...]) + b3_ref[...]

    return pl.pallas_call(
        body,
        grid=(n // blk,),
        in_specs=[
            pl.BlockSpec((blk, d1), lambda i: (i, 0)),
            pl.BlockSpec((d1, d2), lambda i: (0, 0)),
            pl.BlockSpec((1, d2), lambda i: (0, 0)),
            pl.BlockSpec((d2, d3), lambda i: (0, 0)),
            pl.BlockSpec((1, d3), lambda i: (0, 0)),
            pl.BlockSpec((d3, dout), lambda i: (0, 0)),
            pl.BlockSpec((1, dout), lambda i: (0, 0)),
        ],
        out_specs=pl.BlockSpec((blk, dout), lambda i: (i, 0)),
        out_shape=jax.ShapeDtypeStruct((n, dout), jnp.float32),
    )(h, w1, b1.reshape(1, d2), w2, b2.reshape(1, d3), w3, b3.reshape(1, dout))


def kernel(x, edge_index, edge_attr, params):
    n, nfeat = x.shape
    e_total = edge_attr.shape[0]
    src = edge_index[0].astype(jnp.int32)
    dst = edge_index[1].astype(jnp.int32)

    d0 = 16
    h = jnp.pad(x, ((0, 0), (0, d0 - nfeat)))

    bucket, seg = _make_bucket(n, e_total)
    bsrc, bdl, beid, cnts = bucket(src, dst)

    edge16 = _make_edge_aggr(n, e_total, d0, seg)
    edge128 = _make_edge_aggr(n, e_total, 128, seg)

    for i in range(6):
        p = params["conv%d" % i]
        if i == 0:
            we = jnp.pad(p["we"], ((0, 0), (0, d0 - nfeat)))
            be = jnp.pad(p["be"], (0, d0 - nfeat))
            w1 = jnp.pad(p["w1"], ((0, d0 - nfeat), (0, 0)))
            ek = edge16
        else:
            we, be, w1 = p["we"], p["be"], p["w1"]
            ek = edge128
        e = _edge_bias(edge_attr, we, be)
        ag = ek(h, e, bsrc, bdl, beid, cnts)
        h = _node_mlp(h, ag, w1, p["b1"], p["w2"], p["b2"])

    return _head(h, params["lin1_w"], params["lin1_b"],
                 params["lin2_w"], params["lin2_b"],
                 params["lin3_w"], params["lin3_b"])


# speedup vs baseline: 1.2926x; 1.0936x over previous
import functools

import jax
import jax.numpy as jnp
from jax import lax
from jax.experimental import pallas as pl
from jax.experimental.pallas import tpu as pltpu
from jax.experimental.pallas import tpu_sc as plsc

NC = 2
NS = 16
NW = NC * NS
CHUNK = 128
BLK = 16
FLUSH = 2048
BUFC = 3456
TRASH = 313

_MESH = dict(core_axis_name="c", subcore_axis_name="s",
             num_cores=NC, num_subcores=NS)


def _node_lo(w):
    return 312 * w + jnp.minimum(w, 16)


def _make_bucket(n_nodes, n_edges):
    seg = n_edges + BUFC
    scn = 1280
    nscan = n_edges // scn

    @functools.partial(
        pl.kernel,
        mesh=plsc.VectorSubcoreMesh(**_MESH),
        compiler_params=pltpu.CompilerParams(use_tc_tiling_on_sc=False, needs_layout_passes=False),
        out_type=(
            jax.ShapeDtypeStruct((NW * seg,), jnp.int32),
            jax.ShapeDtypeStruct((NW * seg,), jnp.int32),
            jax.ShapeDtypeStruct((NW * seg,), jnp.int32),
            jax.ShapeDtypeStruct((NW * 8 + 16,), jnp.int32),
        ),
        scratch_types=[
            pltpu.VMEM((2, scn), jnp.int32),
            pltpu.VMEM((2, scn), jnp.int32),
            pltpu.VMEM((BUFC,), jnp.int32),
            pltpu.VMEM((BUFC,), jnp.int32),
            pltpu.VMEM((BUFC,), jnp.int32),
            pltpu.VMEM((16,), jnp.int32),
            pltpu.SemaphoreType.DMA,
            pltpu.SemaphoreType.DMA,
        ],
    )
    def bucket_kernel(src_hbm, dst_hbm, bsrc_hbm, bdl_hbm, beid_hbm, cnt_hbm,
                      src_v, dst_v, bs_v, bd_v, be_v, cv, lsem0, lsem1):
        c = lax.axis_index("c")
        s = lax.axis_index("s")
        w = c * NS + s
        lo = _node_lo(w)
        hi = _node_lo(w + 1)
        base = w * seg
        lanes = lax.iota(jnp.int32, 16)

        def flush(written, amt):
            off = pl.multiple_of(base + written, 8)
            pltpu.sync_copy(bs_v.at[pl.ds(0, amt)],
                            bsrc_hbm.at[pl.ds(off, amt)])
            pltpu.sync_copy(bd_v.at[pl.ds(0, amt)],
                            bdl_hbm.at[pl.ds(off, amt)])
            pltpu.sync_copy(be_v.at[pl.ds(0, amt)],
                            beid_hbm.at[pl.ds(off, amt)])

        lsems = (lsem0, lsem1)

        def ld_descs(ci, b):
            coff = pl.ds(pl.multiple_of(ci * scn, 8), scn)
            return (
                pltpu.make_async_copy(src_hbm.at[coff], src_v.at[b], lsems[b]),
                pltpu.make_async_copy(dst_hbm.at[coff], dst_v.at[b], lsems[b]),
            )

        for dsc in ld_descs(0, 0):
            dsc.start()

        def scan_chunk(ci, carry, b):
            ptr_v, written = carry
            for dsc in ld_descs(ci, b):
                dsc.wait()

            @pl.when(ci + 1 < nscan)
            def _():
                for dsc in ld_descs(ci + 1, 1 - b):
                    dsc.start()

            def group(g, ptr_v):
                sl = pl.ds(g * 16, 16)
                dvec = dst_v[b, sl]
                svec = src_v[b, sl]
                eid = ci * scn + g * 16 + lanes
                mask = (dvec >= lo) & (dvec < hi)
                pc = plsc.cumsum(mask.astype(jnp.int32))
                idx = ptr_v + pc - 1
                plsc.store_scatter(bs_v, [idx], svec, mask=mask)
                plsc.store_scatter(bd_v, [idx], dvec - lo, mask=mask)
                plsc.store_scatter(be_v, [idx], eid, mask=mask)
                return ptr_v + plsc.all_reduce_population_count(mask)

            ptr_v = lax.fori_loop(0, scn // 16, group, ptr_v, unroll=2)

            def do_flush(args):
                ptr_v, written = args
                flush(written, FLUSH)
                for b in (bs_v, bd_v, be_v):
                    for t in range((BUFC - FLUSH) // 16):
                        tail = b[pl.ds(FLUSH + t * 16, 16)]
                        b[pl.ds(t * 16, 16)] = tail
                return ptr_v - FLUSH, written + FLUSH

            return lax.cond(ptr_v[0] >= FLUSH, do_flush, lambda a: a,
                            (ptr_v, written))

        def chunk_pair(t, carry):
            carry = scan_chunk(2 * t, carry, 0)
            return scan_chunk(2 * t + 1, carry, 1)

        ptr_v, written = lax.fori_loop(0, nscan // 2, chunk_pair,
                                       (jnp.zeros((16,), jnp.int32),
                                        jnp.int32(0)))
        ptr = ptr_v[0]

        total = written + ptr
        pad = (-total) % CHUNK
        zeros16 = jnp.zeros((16,), jnp.int32)
        trash16 = jnp.full((16,), TRASH, jnp.int32)
        lanes16 = lax.iota(jnp.int32, 16)
        for t in range(CHUNK // 16):
            pidx = ptr + t * 16 + lanes16
            plsc.store_scatter(bs_v, [pidx], zeros16)
            plsc.store_scatter(bd_v, [pidx], trash16)
            plsc.store_scatter(be_v, [pidx], zeros16)
        flush(written, BUFC)
        cv[pl.ds(0, 16)] = jnp.full((16,), total + pad, jnp.int32)
        pltpu.sync_copy(cv.at[pl.ds(0, 8)],
                        cnt_hbm.at[pl.ds(pl.multiple_of(w * 8, 8), 8)])

    return bucket_kernel, seg


def _make_edge_aggr(n_nodes, n_edges, d, seg):
    @functools.partial(
        pl.kernel,
        mesh=plsc.VectorSubcoreMesh(**_MESH),
        compiler_params=pltpu.CompilerParams(use_tc_tiling_on_sc=False, needs_layout_passes=False),
        out_type=jax.ShapeDtypeStruct((n_nodes, d), jnp.float32),
        scratch_types=[
            pltpu.VMEM((2, BLK * CHUNK), jnp.int32),
            pltpu.VMEM((2, BLK * CHUNK), jnp.int32),
            pltpu.VMEM((2, BLK * CHUNK), jnp.int32),
            pltpu.VMEM((2, CHUNK, d), jnp.float32),
            pltpu.VMEM((2, CHUNK, d), jnp.float32),
            pltpu.VMEM((314, d), jnp.float32),
            pltpu.VMEM((NW * 8 + 16,), jnp.int32),
            pltpu.SemaphoreType.DMA,
            pltpu.SemaphoreType.DMA,
            pltpu.SemaphoreType.DMA,
            pltpu.SemaphoreType.DMA,
        ],
    )
    def edge_kernel(h_hbm, e_hbm, bsrc_hbm, bdl_hbm, beid_hbm, cnt_hbm,
                    out_hbm, srcb, dlb, eidb, rows_v, e_v, aggr, cv,
                    bsem0, bsem1, gsem0, gsem1):
        c = lax.axis_index("c")
        s = lax.axis_index("s")
        w = c * NS + s
        lo = _node_lo(w)
        base = w * seg
        bsems = (bsem0, bsem1)
        gsems = (gsem0, gsem1)

        def zbody(i, _):
            for v in range(d // 16):
                aggr[i, pl.ds(v * 16, 16)] = jnp.zeros((16,), jnp.float32)
            return 0
        lax.fori_loop(0, 314, zbody, 0)

        pltpu.sync_copy(cnt_hbm, cv)
        n = cv[pl.ds(w * 8, 16)][0] // CHUNK
        nb = (n + BLK - 1) // BLK

        def blk_descs(B, bb):
            off = pl.ds(pl.multiple_of(base + B * (BLK * CHUNK), 8), BLK * CHUNK)
            return (
                pltpu.make_async_copy(bsrc_hbm.at[off], srcb.at[bb], bsems[bb]),
                pltpu.make_async_copy(bdl_hbm.at[off], dlb.at[bb], bsems[bb]),
                pltpu.make_async_copy(beid_hbm.at[off], eidb.at[bb], bsems[bb]),
            )

        def issue_blk(B, bb):
            for dsc in blk_descs(B, bb):
                dsc.start()

        def wait_blk(B, bb):
            for dsc in blk_descs(B, bb):
                dsc.wait()

        def g_descs(bb, j, g, i):
            isl = pl.ds(pl.multiple_of(j * CHUNK, 8), CHUNK)
            return (
                pltpu.make_async_copy(h_hbm.at[srcb.at[bb, isl]],
                                      rows_v.at[g], gsems[g]),
                pltpu.make_async_copy(e_hbm.at[eidb.at[bb, isl]],
                                      e_v.at[g], gsems[g]),
            )

        def issue_g(bb, j, g, i):
            for dsc in g_descs(bb, j, g, i):
                dsc.start()

        def wait_g(bb, j, g, i):
            for dsc in g_descs(bb, j, g, i):
                dsc.wait()

        @pl.when(nb > 0)
        def _():
            issue_blk(0, 0)

        def block_body(B, bb):
            wait_blk(B, bb)

            @pl.when(B + 1 < nb)
            def _():
                issue_blk(B + 1, 1 - bb)

            @pl.when(B * BLK < n)
            def _():
                issue_g(bb, 0, 0, B * BLK)

            def pair(t, _):
                for parity in range(2):
                    j = 2 * t + parity
                    i = B * BLK + j

                    @pl.when(i < n)
                    def _():
                        @pl.when((j + 1 < BLK) & (i + 1 < n))
                        def _():
                            issue_g(bb, j + 1, 1 - parity, i + 1)
                        wait_g(bb, j, parity, i)

                        def ebody(g16, _):
                            dvec = dlb[bb, pl.ds(j * CHUNK + g16 * 16, 16)]
                            for kk in range(16):
                                dl = dvec[kk]
                                ii = g16 * 16 + kk
                                for v in range(d // 16):
                                    sl = pl.ds(v * 16, 16)
                                    m = jnp.maximum(
                                        rows_v[parity, ii, sl]
                                        + e_v[parity, ii, sl], 0.0)
                                    plsc.addupdate(aggr.at[dl, sl], m)
                            return 0
                        lax.fori_loop(0, CHUNK // 16, ebody, 0)
                return 0
            lax.fori_loop(0, BLK // 2, pair, 0)

        def outer(t, _):
            for parity in range(2):
                B = 2 * t + parity

                @pl.when(B < nb)
                def _():
                    block_body(B, parity)
            return 0
        lax.fori_loop(0, (nb + 1) // 2, outer, 0)

        @pl.when(w < 16)
        def _():
            pltpu.sync_copy(aggr.at[pl.ds(0, 313)],
                            out_hbm.at[pl.ds(lo, 313)])

        @pl.when(w >= 16)
        def _():
            pltpu.sync_copy(aggr.at[pl.ds(0, 312)],
                            out_hbm.at[pl.ds(lo, 312)])

    return edge_kernel


def _dot16(a, b):
    return jnp.dot(a.astype(jnp.bfloat16), b.astype(jnp.bfloat16),
                   preferred_element_type=jnp.float32)


def _edge_bias(edge_attr, we, be):
    e_total, k = edge_attr.shape
    d = we.shape[1]
    be2 = be.reshape(1, d)
    blk = 4000

    def body(a_ref, w_ref, b_ref, o_ref):
        o_ref[...] = _dot16(a_ref[...], w_ref[...]) + b_ref[...]

    return pl.pallas_call(
        body,
        grid=(e_total // blk,),
        in_specs=[
            pl.BlockSpec((blk, k), lambda i: (i, 0)),
            pl.BlockSpec((k, d), lambda i: (0, 0)),
            pl.BlockSpec((1, d), lambda i: (0, 0)),
        ],
        out_specs=pl.BlockSpec((blk, d), lambda i: (i, 0)),
        out_shape=jax.ShapeDtypeStruct((e_total, d), jnp.float32),
    )(edge_attr, we, be2)


def _node_mlp(h, a0, w1, b1, w2, b2):
    n, din = h.shape
    dh = w1.shape[1]
    blk = 1000

    def body(h_ref, a0_ref, w1_ref, b1_ref, w2_ref, b2_ref, o_ref):
        z = h_ref[...] + a0_ref[...]
        t = jnp.maximum(_dot16(z, w1_ref[...]) + b1_ref[...], 0.0)
        o_ref[...] = jnp.maximum(_dot16(t, w2_ref[...]) + b2_ref[...], 0.0)

    return pl.pallas_call(
        body,
        grid=(n // blk,),
        in_specs=[
            pl.BlockSpec((blk, din), lambda i: (i, 0)),
            pl.BlockSpec((blk, din), lambda i: (i, 0)),
            pl.BlockSpec((din, dh), lambda i: (0, 0)),
            pl.BlockSpec((1, dh), lambda i: (0, 0)),
            pl.BlockSpec((dh, dh), lambda i: (0, 0)),
            pl.BlockSpec((1, dh), lambda i: (0, 0)),
        ],
        out_specs=pl.BlockSpec((blk, dh), lambda i: (i, 0)),
        out_shape=jax.ShapeDtypeStruct((n, dh), jnp.float32),
    )(h, a0, w1, b1.reshape(1, dh), w2, b2.reshape(1, dh))


def _head(h, w1, b1, w2, b2, w3, b3):
    n, d1 = h.shape
    d2, d3, dout = w1.shape[1], w2.shape[1], w3.shape[1]
    blk = 1000

    def body(h_ref, w1_ref, b1_ref, w2_ref, b2_ref, w3_ref, b3_ref, o_ref):
        t = jnp.maximum(_dot16(h_ref[...], w1_ref[...]) + b1_ref[...], 0.0)
        t = jnp.maximum(_dot16(t, w2_ref[...]) + b2_ref[...], 0.0)
        o_ref[...] = _dot16(t, w3_ref[...]) + b3_ref[...]

    return pl.pallas_call(
        body,
        grid=(n // blk,),
        in_specs=[
            pl.BlockSpec((blk, d1), lambda i: (i, 0)),
            pl.BlockSpec((d1, d2), lambda i: (0, 0)),
            pl.BlockSpec((1, d2), lambda i: (0, 0)),
            pl.BlockSpec((d2, d3), lambda i: (0, 0)),
            pl.BlockSpec((1, d3), lambda i: (0, 0)),
            pl.BlockSpec((d3, dout), lambda i: (0, 0)),
            pl.BlockSpec((1, dout), lambda i: (0, 0)),
        ],
        out_specs=pl.BlockSpec((blk, dout), lambda i: (i, 0)),
        out_shape=jax.ShapeDtypeStruct((n, dout), jnp.float32),
    )(h, w1, b1.reshape(1, d2), w2, b2.reshape(1, d3), w3, b3.reshape(1, dout))


def kernel(x, edge_index, edge_attr, params):
    n, nfeat = x.shape
    e_total = edge_attr.shape[0]
    src = edge_index[0].astype(jnp.int32)
    dst = edge_index[1].astype(jnp.int32)

    d0 = 16
    h = jnp.pad(x, ((0, 0), (0, d0 - nfeat)))

    bucket, seg = _make_bucket(n, e_total)
    bsrc, bdl, beid, cnts = bucket(src, dst)

    edge16 = _make_edge_aggr(n, e_total, d0, seg)
    edge128 = _make_edge_aggr(n, e_total, 128, seg)

    for i in range(6):
        p = params["conv%d" % i]
        if i == 0:
            we = jnp.pad(p["we"], ((0, 0), (0, d0 - nfeat)))
            be = jnp.pad(p["be"], (0, d0 - nfeat))
            w1 = jnp.pad(p["w1"], ((0, d0 - nfeat), (0, 0)))
            ek = edge16
        else:
            we, be, w1 = p["we"], p["be"], p["w1"]
            ek = edge128
        e = _edge_bias(edge_attr, we, be)
        ag = ek(h, e, bsrc, bdl, beid, cnts)
        h = _node_mlp(h, ag, w1, p["b1"], p["w2"], p["b2"])

    return _head(h, params["lin1_w"], params["lin1_b"],
                 params["lin2_w"], params["lin2_b"],
                 params["lin3_w"], params["lin3_b"])


# speedup vs baseline: 1.3221x; 1.0229x over previous
import functools

import jax
import jax.numpy as jnp
from jax import lax
from jax.experimental import pallas as pl
from jax.experimental.pallas import tpu as pltpu
from jax.experimental.pallas import tpu_sc as plsc

NC = 2
NS = 16
NW = NC * NS
CHUNK = 128
BLK = 16
FLUSH = 2048
BUFC = 3456
TRASH = 313

_MESH = dict(core_axis_name="c", subcore_axis_name="s",
             num_cores=NC, num_subcores=NS)


def _node_lo(w):
    return 312 * w + jnp.minimum(w, 16)


def _make_bucket(n_nodes, n_edges):
    seg = n_edges + BUFC
    scn = 1280
    nscan = n_edges // scn

    @functools.partial(
        pl.kernel,
        mesh=plsc.VectorSubcoreMesh(**_MESH),
        compiler_params=pltpu.CompilerParams(use_tc_tiling_on_sc=False, needs_layout_passes=False),
        out_type=(
            jax.ShapeDtypeStruct((NW * seg,), jnp.int32),
            jax.ShapeDtypeStruct((NW * seg,), jnp.int32),
            jax.ShapeDtypeStruct((NW * seg,), jnp.int32),
            jax.ShapeDtypeStruct((NW * 8 + 16,), jnp.int32),
        ),
        scratch_types=[
            pltpu.VMEM((2, scn), jnp.int32),
            pltpu.VMEM((2, scn), jnp.int32),
            pltpu.VMEM((BUFC,), jnp.int32),
            pltpu.VMEM((BUFC,), jnp.int32),
            pltpu.VMEM((BUFC,), jnp.int32),
            pltpu.VMEM((16,), jnp.int32),
            pltpu.SemaphoreType.DMA,
            pltpu.SemaphoreType.DMA,
        ],
    )
    def bucket_kernel(src_hbm, dst_hbm, bsrc_hbm, bdl_hbm, beid_hbm, cnt_hbm,
                      src_v, dst_v, bs_v, bd_v, be_v, cv, lsem0, lsem1):
        c = lax.axis_index("c")
        s = lax.axis_index("s")
        w = c * NS + s
        lo = _node_lo(w)
        hi = _node_lo(w + 1)
        base = w * seg
        lanes = lax.iota(jnp.int32, 16)

        def flush(written, amt):
            off = pl.multiple_of(base + written, 8)
            pltpu.sync_copy(bs_v.at[pl.ds(0, amt)],
                            bsrc_hbm.at[pl.ds(off, amt)])
            pltpu.sync_copy(bd_v.at[pl.ds(0, amt)],
                            bdl_hbm.at[pl.ds(off, amt)])
            pltpu.sync_copy(be_v.at[pl.ds(0, amt)],
                            beid_hbm.at[pl.ds(off, amt)])

        lsems = (lsem0, lsem1)

        def ld_descs(ci, b):
            coff = pl.ds(pl.multiple_of(ci * scn, 8), scn)
            return (
                pltpu.make_async_copy(src_hbm.at[coff], src_v.at[b], lsems[b]),
                pltpu.make_async_copy(dst_hbm.at[coff], dst_v.at[b], lsems[b]),
            )

        for dsc in ld_descs(0, 0):
            dsc.start()

        def scan_chunk(ci, carry, b):
            ptr_v, written = carry
            for dsc in ld_descs(ci, b):
                dsc.wait()

            @pl.when(ci + 1 < nscan)
            def _():
                for dsc in ld_descs(ci + 1, 1 - b):
                    dsc.start()

            def group(g, ptr_v):
                sl = pl.ds(g * 16, 16)
                dvec = dst_v[b, sl]
                svec = src_v[b, sl]
                eid = ci * scn + g * 16 + lanes
                mask = (dvec >= lo) & (dvec < hi)
                pc = plsc.cumsum(mask.astype(jnp.int32))
                idx = ptr_v + pc - 1
                plsc.store_scatter(bs_v, [idx], svec, mask=mask)
                plsc.store_scatter(bd_v, [idx], dvec - lo, mask=mask)
                plsc.store_scatter(be_v, [idx], eid, mask=mask)
                return ptr_v + plsc.all_reduce_population_count(mask)

            ptr_v = lax.fori_loop(0, scn // 16, group, ptr_v, unroll=2)

            def do_flush(args):
                ptr_v, written = args
                flush(written, FLUSH)
                for b in (bs_v, bd_v, be_v):
                    for t in range((BUFC - FLUSH) // 16):
                        tail = b[pl.ds(FLUSH + t * 16, 16)]
                        b[pl.ds(t * 16, 16)] = tail
                return ptr_v - FLUSH, written + FLUSH

            return lax.cond(ptr_v[0] >= FLUSH, do_flush, lambda a: a,
                            (ptr_v, written))

        def chunk_pair(t, carry):
            carry = scan_chunk(2 * t, carry, 0)
            return scan_chunk(2 * t + 1, carry, 1)

        ptr_v, written = lax.fori_loop(0, nscan // 2, chunk_pair,
                                       (jnp.zeros((16,), jnp.int32),
                                        jnp.int32(0)))
        ptr = ptr_v[0]

        total = written + ptr
        pad = (-total) % CHUNK
        zeros16 = jnp.zeros((16,), jnp.int32)
        trash16 = jnp.full((16,), TRASH, jnp.int32)
        lanes16 = lax.iota(jnp.int32, 16)
        for t in range(CHUNK // 16):
            pidx = ptr + t * 16 + lanes16
            plsc.store_scatter(bs_v, [pidx], zeros16)
            plsc.store_scatter(bd_v, [pidx], trash16)
            plsc.store_scatter(be_v, [pidx], zeros16)
        flush(written, BUFC)
        cv[pl.ds(0, 16)] = jnp.full((16,), total + pad, jnp.int32)
        pltpu.sync_copy(cv.at[pl.ds(0, 8)],
                        cnt_hbm.at[pl.ds(pl.multiple_of(w * 8, 8), 8)])

    return bucket_kernel, seg


def _make_edge_aggr(n_nodes, n_edges, d, seg):
    @functools.partial(
        pl.kernel,
        mesh=plsc.VectorSubcoreMesh(**_MESH),
        compiler_params=pltpu.CompilerParams(use_tc_tiling_on_sc=False, needs_layout_passes=False),
        out_type=jax.ShapeDtypeStruct((n_nodes, d), jnp.float32),
        scratch_types=[
            pltpu.VMEM((2, BLK * CHUNK), jnp.int32),
            pltpu.VMEM((2, BLK * CHUNK), jnp.int32),
            pltpu.VMEM((2, BLK * CHUNK), jnp.int32),
            pltpu.VMEM((2, CHUNK, d), jnp.float32),
            pltpu.VMEM((2, CHUNK, d), jnp.float32),
            pltpu.VMEM((314, d), jnp.float32),
            pltpu.VMEM((NW * 8 + 16,), jnp.int32),
            pltpu.SemaphoreType.DMA,
            pltpu.SemaphoreType.DMA,
            pltpu.SemaphoreType.DMA,
            pltpu.SemaphoreType.DMA,
        ],
    )
    def edge_kernel(h_hbm, e_hbm, bsrc_hbm, bdl_hbm, beid_hbm, cnt_hbm,
                    out_hbm, srcb, dlb, eidb, rows_v, e_v, aggr, cv,
                    bsem0, bsem1, gsem0, gsem1):
        c = lax.axis_index("c")
        s = lax.axis_index("s")
        w = c * NS + s
        lo = _node_lo(w)
        base = w * seg
        bsems = (bsem0, bsem1)
        gsems = (gsem0, gsem1)

        def zbody(i, _):
            for v in range(d // 16):
                aggr[i, pl.ds(v * 16, 16)] = jnp.zeros((16,), jnp.float32)
            return 0
        lax.fori_loop(0, 314, zbody, 0)

        pltpu.sync_copy(cnt_hbm, cv)
        n = cv[pl.ds(w * 8, 16)][0] // CHUNK
        nb = (n + BLK - 1) // BLK

        def blk_descs(B, bb):
            off = pl.ds(pl.multiple_of(base + B * (BLK * CHUNK), 8), BLK * CHUNK)
            return (
                pltpu.make_async_copy(bsrc_hbm.at[off], srcb.at[bb], bsems[bb]),
                pltpu.make_async_copy(bdl_hbm.at[off], dlb.at[bb], bsems[bb]),
                pltpu.make_async_copy(beid_hbm.at[off], eidb.at[bb], bsems[bb]),
            )

        def issue_blk(B, bb):
            for dsc in blk_descs(B, bb):
                dsc.start()

        def wait_blk(B, bb):
            for dsc in blk_descs(B, bb):
                dsc.wait()

        def g_descs(bb, j, g, i):
            isl = pl.ds(pl.multiple_of(j * CHUNK, 8), CHUNK)
            return (
                pltpu.make_async_copy(h_hbm.at[srcb.at[bb, isl]],
                                      rows_v.at[g], gsems[g]),
                pltpu.make_async_copy(e_hbm.at[eidb.at[bb, isl]],
                                      e_v.at[g], gsems[g]),
            )

        def issue_g(bb, j, g, i):
            for dsc in g_descs(bb, j, g, i):
                dsc.start()

        def wait_g(bb, j, g, i):
            for dsc in g_descs(bb, j, g, i):
                dsc.wait()

        @pl.when(nb > 0)
        def _():
            issue_blk(0, 0)
            wait_blk(0, 0)

        @pl.when(n > 0)
        def _():
            issue_g(0, 0, 0, 0)

        def block_body(B, bb):
            @pl.when(B + 1 < nb)
            def _():
                issue_blk(B + 1, 1 - bb)

            def pair(t, _):
                for parity in range(2):
                    j = 2 * t + parity
                    i = B * BLK + j

                    @pl.when(i < n)
                    def _():
                        @pl.when((j + 1 < BLK) & (i + 1 < n))
                        def _():
                            issue_g(bb, j + 1, 1 - parity, i + 1)

                        @pl.when((j + 1 >= BLK) & (i + 1 < n))
                        def _():
                            wait_blk(B + 1, 1 - bb)
                            issue_g(1 - bb, 0, 1 - parity, i + 1)
                        wait_g(bb, j, parity, i)

                        def ebody(g16, _):
                            dvec = dlb[bb, pl.ds(j * CHUNK + g16 * 16, 16)]
                            for kk in range(16):
                                dl = dvec[kk]
                                ii = g16 * 16 + kk
                                for v in range(d // 16):
                                    sl = pl.ds(v * 16, 16)
                                    m = jnp.maximum(
                                        rows_v[parity, ii, sl]
                                        + e_v[parity, ii, sl], 0.0)
                                    plsc.addupdate(aggr.at[dl, sl], m)
                            return 0
                        lax.fori_loop(0, CHUNK // 16, ebody, 0)
                return 0
            lax.fori_loop(0, BLK // 2, pair, 0)

        def outer(t, _):
            for parity in range(2):
                B = 2 * t + parity

                @pl.when(B < nb)
                def _():
                    block_body(B, parity)
            return 0
        lax.fori_loop(0, (nb + 1) // 2, outer, 0)

        @pl.when(w < 16)
        def _():
            pltpu.sync_copy(aggr.at[pl.ds(0, 313)],
                            out_hbm.at[pl.ds(lo, 313)])

        @pl.when(w >= 16)
        def _():
            pltpu.sync_copy(aggr.at[pl.ds(0, 312)],
                            out_hbm.at[pl.ds(lo, 312)])

    return edge_kernel


def _dot16(a, b):
    return jnp.dot(a.astype(jnp.bfloat16), b.astype(jnp.bfloat16),
                   preferred_element_type=jnp.float32)


def _edge_bias(edge_attr, we, be):
    e_total, k = edge_attr.shape
    d = we.shape[1]
    be2 = be.reshape(1, d)
    blk = 4000

    def body(a_ref, w_ref, b_ref, o_ref):
        o_ref[...] = _dot16(a_ref[...], w_ref[...]) + b_ref[...]

    return pl.pallas_call(
        body,
        grid=(e_total // blk,),
        in_specs=[
            pl.BlockSpec((blk, k), lambda i: (i, 0)),
            pl.BlockSpec((k, d), lambda i: (0, 0)),
            pl.BlockSpec((1, d), lambda i: (0, 0)),
        ],
        out_specs=pl.BlockSpec((blk, d), lambda i: (i, 0)),
        out_shape=jax.ShapeDtypeStruct((e_total, d), jnp.float32),
    )(edge_attr, we, be2)


def _node_mlp(h, a0, w1, b1, w2, b2):
    n, din = h.shape
    dh = w1.shape[1]
    blk = 1000

    def body(h_ref, a0_ref, w1_ref, b1_ref, w2_ref, b2_ref, o_ref):
        z = h_ref[...] + a0_ref[...]
        t = jnp.maximum(_dot16(z, w1_ref[...]) + b1_ref[...], 0.0)
        o_ref[...] = jnp.maximum(_dot16(t, w2_ref[...]) + b2_ref[...], 0.0)

    return pl.pallas_call(
        body,
        grid=(n // blk,),
        in_specs=[
            pl.BlockSpec((blk, din), lambda i: (i, 0)),
            pl.BlockSpec((blk, din), lambda i: (i, 0)),
            pl.BlockSpec((din, dh), lambda i: (0, 0)),
            pl.BlockSpec((1, dh), lambda i: (0, 0)),
            pl.BlockSpec((dh, dh), lambda i: (0, 0)),
            pl.BlockSpec((1, dh), lambda i: (0, 0)),
        ],
        out_specs=pl.BlockSpec((blk, dh), lambda i: (i, 0)),
        out_shape=jax.ShapeDtypeStruct((n, dh), jnp.float32),
    )(h, a0, w1, b1.reshape(1, dh), w2, b2.reshape(1, dh))


def _head(h, w1, b1, w2, b2, w3, b3):
    n, d1 = h.shape
    d2, d3, dout = w1.shape[1], w2.shape[1], w3.shape[1]
    blk = 1000

    def body(h_ref, w1_ref, b1_ref, w2_ref, b2_ref, w3_ref, b3_ref, o_ref):
        t = jnp.maximum(_dot16(h_ref[...], w1_ref[...]) + b1_ref[...], 0.0)
        t = jnp.maximum(_dot16(t, w2_ref[...]) + b2_ref[...], 0.0)
        o_ref[...] = _dot16(t, w3_ref[...]) + b3_ref[...]

    return pl.pallas_call(
        body,
        grid=(n // blk,),
        in_specs=[
            pl.BlockSpec((blk, d1), lambda i: (i, 0)),
            pl.BlockSpec((d1, d2), lambda i: (0, 0)),
            pl.BlockSpec((1, d2), lambda i: (0, 0)),
            pl.BlockSpec((d2, d3), lambda i: (0, 0)),
            pl.BlockSpec((1, d3), lambda i: (0, 0)),
            pl.BlockSpec((d3, dout), lambda i: (0, 0)),
            pl.BlockSpec((1, dout), lambda i: (0, 0)),
        ],
        out_specs=pl.BlockSpec((blk, dout), lambda i: (i, 0)),
        out_shape=jax.ShapeDtypeStruct((n, dout), jnp.float32),
    )(h, w1, b1.reshape(1, d2), w2, b2.reshape(1, d3), w3, b3.reshape(1, dout))


def kernel(x, edge_index, edge_attr, params):
    n, nfeat = x.shape
    e_total = edge_attr.shape[0]
    src = edge_index[0].astype(jnp.int32)
    dst = edge_index[1].astype(jnp.int32)

    d0 = 16
    h = jnp.pad(x, ((0, 0), (0, d0 - nfeat)))

    bucket, seg = _make_bucket(n, e_total)
    bsrc, bdl, beid, cnts = bucket(src, dst)

    edge16 = _make_edge_aggr(n, e_total, d0, seg)
    edge128 = _make_edge_aggr(n, e_total, 128, seg)

    for i in range(6):
        p = params["conv%d" % i]
        if i == 0:
            we = jnp.pad(p["we"], ((0, 0), (0, d0 - nfeat)))
            be = jnp.pad(p["be"], (0, d0 - nfeat))
            w1 = jnp.pad(p["w1"], ((0, d0 - nfeat), (0, 0)))
            ek = edge16
        else:
            we, be, w1 = p["we"], p["be"], p["w1"]
            ek = edge128
        e = _edge_bias(edge_attr, we, be)
        ag = ek(h, e, bsrc, bdl, beid, cnts)
        h = _node_mlp(h, ag, w1, p["b1"], p["w2"], p["b2"])

    return _head(h, params["lin1_w"], params["lin1_b"],
                 params["lin2_w"], params["lin2_b"],
                 params["lin3_w"], params["lin3_b"])
